# Initial kernel scaffold; baseline (speedup 1.0000x reference)
import jax
import jax.numpy as jnp
from jax.experimental import pallas as pl


def kernel(node_features, incidence_nodes, incidence_edges, perturbation_mask, W, b):
    raise NotImplementedError("write your pallas kernel here")


# speedup vs baseline: 6.1878x; 6.1878x over previous
import functools

import jax
import jax.numpy as jnp
from jax import lax
from jax.experimental import pallas as pl
from jax.experimental.pallas import tpu as pltpu
from jax.experimental.pallas import tpu_sc as plsc

N = 10000
E = 2000
INC = 320000
D = 128

N_PAD = 10240
E_PAD = 2048
CHUNK = 128
NC, NS = 2, 16
NW = NC * NS
INC_PAD = 323584
PAIRS_PER_TILE_ALL = INC_PAD // NS
CHUNKS_ALL = PAIRS_PER_TILE_ALL // CHUNK
PAIRS_PER_TILE_HALF = INC_PAD // NW
CHUNKS_HALF = PAIRS_PER_TILE_HALF // CHUNK

_mesh = plsc.VectorSubcoreMesh(core_axis_name="c", subcore_axis_name="s")
_sc_params = pltpu.CompilerParams(needs_layout_passes=False)


def _zero_1d(ref, nwords):
    def body(i, _):
        ref[pl.ds(i * 16, 16)] = jnp.zeros((16,), jnp.float32)
        return 0
    lax.fori_loop(0, nwords // 16, body, 0)


def _zero_2d(ref, rows, cols):
    def body(i, _):
        r = i // (cols // 16)
        c = (i % (cols // 16)) * 16
        ref[r, pl.ds(c, 16)] = jnp.zeros((16,), jnp.float32)
        return 0
    lax.fori_loop(0, rows * (cols // 16), body, 0)


def _vadd_1d(dst, src, nwords):
    def body(i, _):
        s = pl.ds(i * 16, 16)
        dst[s] = dst[s] + src[s]
        return 0
    lax.fori_loop(0, nwords // 16, body, 0)


def _tree_reduce(sid, vec, tmp, shared, nwords):
    pltpu.sync_copy(vec, shared.at[sid])
    plsc.subcore_barrier()
    for r in (8, 4, 2, 1):
        @pl.when(sid < r)
        def _():
            pltpu.sync_copy(shared.at[sid + r], tmp)
            _vadd_1d(vec, tmp, nwords)
            pltpu.sync_copy(vec, shared.at[sid])
        plsc.subcore_barrier()


@functools.partial(
    pl.kernel,
    out_type=(
        jax.ShapeDtypeStruct((NC * E_PAD, D), jnp.float32),
        jax.ShapeDtypeStruct((N_PAD,), jnp.float32),
        jax.ShapeDtypeStruct((E_PAD,), jnp.float32),
        jax.ShapeDtypeStruct((E_PAD,), jnp.float32),
    ),
    mesh=_mesh,
    scratch_types=(
        pltpu.VMEM((CHUNK,), jnp.int32),
        pltpu.VMEM((CHUNK,), jnp.int32),
        pltpu.VMEM((CHUNK, D), jnp.float32),
        pltpu.VMEM((N_PAD,), jnp.float32),
        pltpu.VMEM((N_PAD,), jnp.float32),
        pltpu.VMEM((E_PAD,), jnp.float32),
        pltpu.VMEM((E_PAD,), jnp.float32),
        pltpu.VMEM_SHARED((NS, N_PAD), jnp.float32),
        pltpu.VMEM_SHARED((NS, E_PAD), jnp.float32),
        pltpu.VMEM_SHARED((E_PAD, D), jnp.float32),
        pltpu.SemaphoreType.DMA,
    ),
    compiler_params=_sc_params,
)
def _k1(xt_hbm, nidx_hbm, eidx_hbm,
        esum_out, deg_out, cnt_out, den_out,
        nidx_v, eidx_v, rows_v, deg_v, deg_tmp, ev_v, ev_tmp,
        sh_deg, sh_edge, e_acc, sem):
    cid = lax.axis_index("c")
    sid = lax.axis_index("s")
    ones = jnp.ones((16,), jnp.float32)

    _zero_1d(deg_v, N_PAD)
    _zero_1d(ev_v, E_PAD)
    _zero_2d(rows_v, CHUNK, D)
    pltpu.sync_copy(rows_v, e_acc.at[pl.ds(sid * CHUNK, CHUNK)])

    base_all = sid * PAIRS_PER_TILE_ALL

    def chunk_a(ci, _):
        b = base_all + ci * CHUNK
        pltpu.sync_copy(nidx_hbm.at[pl.ds(b, CHUNK)], nidx_v)
        pltpu.sync_copy(eidx_hbm.at[pl.ds(b, CHUNK)], eidx_v)
        for j in range(CHUNK // 16):
            nv = nidx_v[pl.ds(j * 16, 16)]
            evi = eidx_v[pl.ds(j * 16, 16)]
            plsc.addupdate_scatter(deg_v, [nv], ones)
            plsc.addupdate_scatter(ev_v, [evi], ones)
        return 0
    lax.fori_loop(0, CHUNKS_ALL, chunk_a, 0)

    _tree_reduce(sid, deg_v, deg_tmp, sh_deg, N_PAD)
    _tree_reduce(sid, ev_v, ev_tmp, sh_edge, E_PAD)

    @pl.when(jnp.logical_and(cid == 0, sid == 0))
    def _():
        pltpu.sync_copy(deg_v, deg_out)
        pltpu.sync_copy(ev_v, cnt_out)

    plsc.subcore_barrier()
    pltpu.sync_copy(sh_deg.at[0], deg_v)

    _zero_1d(ev_v, E_PAD)

    def chunk_b(ci, _):
        b = base_all + ci * CHUNK
        pltpu.sync_copy(nidx_hbm.at[pl.ds(b, CHUNK)], nidx_v)
        pltpu.sync_copy(eidx_hbm.at[pl.ds(b, CHUNK)], eidx_v)
        for j in range(CHUNK // 16):
            nv = nidx_v[pl.ds(j * 16, 16)]
            evi = eidx_v[pl.ds(j * 16, 16)]
            dv = plsc.load_gather(deg_v, [nv])
            plsc.addupdate_scatter(ev_v, [evi], dv)
        return 0
    lax.fori_loop(0, CHUNKS_ALL, chunk_b, 0)

    _tree_reduce(sid, ev_v, ev_tmp, sh_edge, E_PAD)

    @pl.when(jnp.logical_and(cid == 0, sid == 0))
    def _():
        pltpu.sync_copy(ev_v, den_out)

    base_half = cid * (INC_PAD // NC) + sid * PAIRS_PER_TILE_HALF

    def chunk_c(ci, _):
        b = base_half + ci * CHUNK
        pltpu.sync_copy(nidx_hbm.at[pl.ds(b, CHUNK)], nidx_v)
        pltpu.sync_copy(eidx_hbm.at[pl.ds(b, CHUNK)], eidx_v)
        pltpu.async_copy(xt_hbm.at[nidx_v], rows_v, sem).wait()
        pltpu.sync_copy(rows_v, e_acc.at[eidx_v], add=True)
        return 0
    lax.fori_loop(0, CHUNKS_HALF, chunk_c, 0)

    plsc.subcore_barrier()

    pltpu.sync_copy(e_acc.at[pl.ds(sid * CHUNK, CHUNK)], rows_v)
    pltpu.sync_copy(rows_v, esum_out.at[pl.ds(cid * E_PAD + sid * CHUNK, CHUNK)])


ROWS_PER_TILE = N_PAD // NS
STAGE_STEPS = ROWS_PER_TILE // CHUNK


@functools.partial(
    pl.kernel,
    out_type=jax.ShapeDtypeStruct((NC * N_PAD, D), jnp.float32),
    mesh=_mesh,
    scratch_types=(
        pltpu.VMEM((CHUNK,), jnp.int32),
        pltpu.VMEM((CHUNK,), jnp.int32),
        pltpu.VMEM((CHUNK, D), jnp.float32),
        pltpu.VMEM_SHARED((N_PAD, D), jnp.float32),
        pltpu.SemaphoreType.DMA,
    ),
    compiler_params=_sc_params,
)
def _k4(msg_hbm, nidx_hbm, eidx_hbm, out_hbm,
        nidx_v, eidx_v, rows_v, out_acc, sem):
    cid = lax.axis_index("c")
    sid = lax.axis_index("s")

    _zero_2d(rows_v, CHUNK, D)
    for j in range(STAGE_STEPS):
        pltpu.sync_copy(
            rows_v, out_acc.at[pl.ds(sid * ROWS_PER_TILE + j * CHUNK, CHUNK)])
    plsc.subcore_barrier()

    base = cid * (INC_PAD // NC) + sid * PAIRS_PER_TILE_HALF

    def chunk(ci, _):
        b = base + ci * CHUNK
        pltpu.sync_copy(nidx_hbm.at[pl.ds(b, CHUNK)], nidx_v)
        pltpu.sync_copy(eidx_hbm.at[pl.ds(b, CHUNK)], eidx_v)
        pltpu.async_copy(msg_hbm.at[eidx_v], rows_v, sem).wait()
        pltpu.sync_copy(rows_v, out_acc.at[nidx_v], add=True)
        return 0
    lax.fori_loop(0, CHUNKS_HALF, chunk, 0)

    plsc.subcore_barrier()

    for j in range(STAGE_STEPS):
        r = sid * ROWS_PER_TILE + j * CHUNK
        pltpu.sync_copy(out_acc.at[pl.ds(r, CHUNK)], rows_v)
        pltpu.sync_copy(rows_v, out_hbm.at[pl.ds(cid * N_PAD + r, CHUNK)])


def _k0_body(x_ref, s_ref, w_ref, b_ref, o_ref):
    xs = x_ref[...] * s_ref[...]
    o_ref[...] = jnp.dot(xs, w_ref[...],
                         preferred_element_type=jnp.float32) + b_ref[...]


def _k3_body(esum_ref, cnt_ref, den_ref, deg_ref, msg_ref, isd_ref):
    esum = esum_ref[0] + esum_ref[1]
    cnt_safe = jnp.maximum(cnt_ref[...], 1.0)
    h = esum / cnt_safe
    d_e = jnp.maximum(den_ref[...] / cnt_safe, 1.0)
    msg_ref[...] = h * lax.rsqrt(d_e)
    isd_ref[...] = lax.rsqrt(jnp.maximum(deg_ref[...], 1.0))


def _k5_body(p_ref, isd_ref, o_ref):
    o_ref[...] = (p_ref[0] + p_ref[1]) * isd_ref[...]


def kernel(node_features, incidence_nodes, incidence_edges,
           perturbation_mask, W, b):
    f32 = jnp.float32
    x_pad = jnp.pad(node_features, ((0, N_PAD - N), (0, 0)))
    scale = (~perturbation_mask).astype(f32)[:, None]
    scale_pad = jnp.pad(scale, ((0, N_PAD - N), (0, 0)))
    b2 = b.reshape(1, D)

    pad = INC_PAD - INC
    nidx = jnp.concatenate(
        [incidence_nodes.astype(jnp.int32), jnp.full((pad,), N, jnp.int32)])
    eidx = jnp.concatenate(
        [incidence_edges.astype(jnp.int32), jnp.full((pad,), E, jnp.int32)])

    xt_pad = pl.pallas_call(
        _k0_body,
        grid=(5,),
        in_specs=[
            pl.BlockSpec((2048, D), lambda i: (i, 0)),
            pl.BlockSpec((2048, 1), lambda i: (i, 0)),
            pl.BlockSpec((D, D), lambda i: (0, 0)),
            pl.BlockSpec((1, D), lambda i: (0, 0)),
        ],
        out_specs=pl.BlockSpec((2048, D), lambda i: (i, 0)),
        out_shape=jax.ShapeDtypeStruct((N_PAD, D), f32),
    )(x_pad, scale_pad, W, b2)

    esum_p, deg, cnt, den = _k1(xt_pad, nidx, eidx)
    esum_p = esum_p.reshape(NC, E_PAD, D)

    msg, isd = pl.pallas_call(
        _k3_body,
        grid=(1,),
        in_specs=[
            pl.BlockSpec((NC, E_PAD, D), lambda i: (0, 0, 0)),
            pl.BlockSpec((E_PAD, 1), lambda i: (0, 0)),
            pl.BlockSpec((E_PAD, 1), lambda i: (0, 0)),
            pl.BlockSpec((N_PAD, 1), lambda i: (0, 0)),
        ],
        out_specs=[
            pl.BlockSpec((E_PAD, D), lambda i: (0, 0)),
            pl.BlockSpec((N_PAD, 1), lambda i: (0, 0)),
        ],
        out_shape=[
            jax.ShapeDtypeStruct((E_PAD, D), f32),
            jax.ShapeDtypeStruct((N_PAD, 1), f32),
        ],
    )(esum_p, cnt[:, None], den[:, None], deg[:, None])

    out_p = _k4(msg, nidx, eidx).reshape(NC, N_PAD, D)

    out = pl.pallas_call(
        _k5_body,
        grid=(5,),
        in_specs=[
            pl.BlockSpec((NC, 2000, D), lambda i: (0, i, 0)),
            pl.BlockSpec((2000, 1), lambda i: (i, 0)),
        ],
        out_specs=pl.BlockSpec((2000, D), lambda i: (i, 0)),
        out_shape=jax.ShapeDtypeStruct((N, D), f32),
    )(out_p, isd)
    return out


# speedup vs baseline: 6.3427x; 1.0250x over previous
import functools

import jax
import jax.numpy as jnp
from jax import lax
from jax.experimental import pallas as pl
from jax.experimental.pallas import tpu as pltpu
from jax.experimental.pallas import tpu_sc as plsc

N = 10000
E = 2000
INC = 320000
D = 128

N_PAD = 10240
E_PAD = 2048
CHUNK = 128
NC, NS = 2, 16
NW = NC * NS
INC_PAD = 327680
PAIRS_ALL = INC_PAD // NS
CHUNKS_ALL = PAIRS_ALL // CHUNK
PAIRS_HALF = INC_PAD // NW
CHUNKS_HALF = PAIRS_HALF // CHUNK

_mesh = plsc.VectorSubcoreMesh(core_axis_name="c", subcore_axis_name="s")
_sc_params = pltpu.CompilerParams(needs_layout_passes=False)


def _zero_1d(ref, nwords):
    def body(i, _):
        ref[pl.ds(i * 16, 16)] = jnp.zeros((16,), jnp.float32)
        return 0
    lax.fori_loop(0, nwords // 16, body, 0)


def _zero_2d(ref, rows, cols):
    def body(i, _):
        r = i // (cols // 16)
        c = (i % (cols // 16)) * 16
        ref[r, pl.ds(c, 16)] = jnp.zeros((16,), jnp.float32)
        return 0
    lax.fori_loop(0, rows * (cols // 16), body, 0)


def _vadd_1d(dst, src, nwords):
    def body(i, _):
        s = pl.ds(i * 16, 16)
        dst[s] = dst[s] + src[s]
        return 0
    lax.fori_loop(0, nwords // 16, body, 0)


def _tree_reduce(sid, vec, tmp, shared, nwords):
    pltpu.sync_copy(vec, shared.at[sid])
    plsc.subcore_barrier()
    for r in (8, 4, 2, 1):
        @pl.when(sid < r)
        def _():
            pltpu.sync_copy(shared.at[sid + r], tmp)
            _vadd_1d(vec, tmp, nwords)
            pltpu.sync_copy(vec, shared.at[sid])
        plsc.subcore_barrier()


def _gs_pipeline(src_hbm, nidx_hbm, eidx_hbm, base, nchunks,
                 na, ea, nb, eb, rows_a, rows_b,
                 sia, sib, sga, sgb, acc):
    assert nchunks % 2 == 0

    def ld(n_bank, e_bank, sem, c):
        off = base + c * CHUNK
        pltpu.async_copy(nidx_hbm.at[pl.ds(off, CHUNK)], n_bank, sem)
        pltpu.async_copy(eidx_hbm.at[pl.ds(off, CHUNK)], e_bank, sem)

    def ld_wait(n_bank, e_bank, sem, c):
        off = base + c * CHUNK
        pltpu.make_async_copy(nidx_hbm.at[pl.ds(off, CHUNK)], n_bank, sem).wait()
        pltpu.make_async_copy(eidx_hbm.at[pl.ds(off, CHUNK)], e_bank, sem).wait()

    ld(na, ea, sia, 0)
    ld_wait(na, ea, sia, 0)
    pltpu.async_copy(src_hbm.at[na], rows_a, sga)
    ld(nb, eb, sib, 1)

    def body(i2, _):
        c0 = 2 * i2
        c1 = c0 + 1

        ld_wait(nb, eb, sib, c1)
        pltpu.async_copy(src_hbm.at[nb], rows_b, sgb)

        pltpu.make_async_copy(src_hbm.at[na], rows_a, sga).wait()
        pltpu.sync_copy(rows_a, acc.at[ea], add=True)

        @pl.when(c0 + 2 < nchunks)
        def _():
            ld(na, ea, sia, c0 + 2)
            ld_wait(na, ea, sia, c0 + 2)
            pltpu.async_copy(src_hbm.at[na], rows_a, sga)

        pltpu.make_async_copy(src_hbm.at[nb], rows_b, sgb).wait()
        pltpu.sync_copy(rows_b, acc.at[eb], add=True)

        @pl.when(c1 + 2 < nchunks)
        def _():
            ld(nb, eb, sib, c1 + 2)
        return 0
    lax.fori_loop(0, nchunks // 2, body, 0)


@functools.partial(
    pl.kernel,
    out_type=(
        jax.ShapeDtypeStruct((NC * E_PAD, D), jnp.float32),
        jax.ShapeDtypeStruct((N_PAD,), jnp.float32),
        jax.ShapeDtypeStruct((E_PAD,), jnp.float32),
        jax.ShapeDtypeStruct((E_PAD,), jnp.float32),
    ),
    mesh=_mesh,
    scratch_types=(
        pltpu.VMEM((PAIRS_ALL,), jnp.int32),
        pltpu.VMEM((PAIRS_ALL,), jnp.int32),
        pltpu.VMEM((CHUNK,), jnp.int32),
        pltpu.VMEM((CHUNK,), jnp.int32),
        pltpu.VMEM((CHUNK,), jnp.int32),
        pltpu.VMEM((CHUNK,), jnp.int32),
        pltpu.VMEM((CHUNK, D), jnp.float32),
        pltpu.VMEM((CHUNK, D), jnp.float32),
        pltpu.VMEM((N_PAD,), jnp.float32),
        pltpu.VMEM((N_PAD,), jnp.float32),
        pltpu.VMEM((E_PAD,), jnp.float32),
        pltpu.VMEM((E_PAD,), jnp.float32),
        pltpu.VMEM_SHARED((NS, N_PAD), jnp.float32),
        pltpu.VMEM_SHARED((NS, E_PAD), jnp.float32),
        pltpu.VMEM_SHARED((E_PAD, D), jnp.float32),
        pltpu.SemaphoreType.DMA,
        pltpu.SemaphoreType.DMA,
        pltpu.SemaphoreType.DMA,
        pltpu.SemaphoreType.DMA,
    ),
    compiler_params=_sc_params,
)
def _k1(xt_hbm, nidx_hbm, eidx_hbm,
        esum_out, deg_out, cnt_out, den_out,
        nidx_all, eidx_all, na, ea, nb, eb, rows_a, rows_b,
        deg_v, deg_tmp, ev_v, ev_tmp,
        sh_deg, sh_edge, e_acc, sia, sib, sga, sgb):
    cid = lax.axis_index("c")
    sid = lax.axis_index("s")
    ones = jnp.ones((16,), jnp.float32)

    pltpu.sync_copy(nidx_hbm.at[pl.ds(sid * PAIRS_ALL, PAIRS_ALL)], nidx_all)
    pltpu.sync_copy(eidx_hbm.at[pl.ds(sid * PAIRS_ALL, PAIRS_ALL)], eidx_all)

    _zero_1d(deg_v, N_PAD)
    _zero_1d(ev_v, E_PAD)
    _zero_2d(rows_a, CHUNK, D)
    pltpu.sync_copy(rows_a, e_acc.at[pl.ds(sid * CHUNK, CHUNK)])

    def vec_a(k, _):
        nv = nidx_all[pl.ds(k * 16, 16)]
        evi = eidx_all[pl.ds(k * 16, 16)]
        plsc.addupdate_scatter(deg_v, [nv], ones)
        plsc.addupdate_scatter(ev_v, [evi], ones)
        return 0
    lax.fori_loop(0, PAIRS_ALL // 16, vec_a, 0)

    _tree_reduce(sid, deg_v, deg_tmp, sh_deg, N_PAD)
    _tree_reduce(sid, ev_v, ev_tmp, sh_edge, E_PAD)

    @pl.when(jnp.logical_and(cid == 0, sid == 0))
    def _():
        pltpu.sync_copy(deg_v, deg_out)
        pltpu.sync_copy(ev_v, cnt_out)

    plsc.subcore_barrier()
    pltpu.sync_copy(sh_deg.at[0], deg_v)

    _zero_1d(ev_v, E_PAD)

    def vec_b(k, _):
        nv = nidx_all[pl.ds(k * 16, 16)]
        evi = eidx_all[pl.ds(k * 16, 16)]
        dv = plsc.load_gather(deg_v, [nv])
        plsc.addupdate_scatter(ev_v, [evi], dv)
        return 0
    lax.fori_loop(0, PAIRS_ALL // 16, vec_b, 0)

    _tree_reduce(sid, ev_v, ev_tmp, sh_edge, E_PAD)

    @pl.when(jnp.logical_and(cid == 0, sid == 0))
    def _():
        pltpu.sync_copy(ev_v, den_out)

    base = cid * (INC_PAD // NC) + sid * PAIRS_HALF
    _gs_pipeline(xt_hbm, nidx_hbm, eidx_hbm, base, CHUNKS_HALF,
                 na, ea, nb, eb, rows_a, rows_b, sia, sib, sga, sgb, e_acc)

    plsc.subcore_barrier()

    pltpu.sync_copy(e_acc.at[pl.ds(sid * CHUNK, CHUNK)], rows_a)
    pltpu.sync_copy(rows_a, esum_out.at[pl.ds(cid * E_PAD + sid * CHUNK, CHUNK)])


ROWS_PER_TILE = N_PAD // NS
STAGE_STEPS = ROWS_PER_TILE // CHUNK


@functools.partial(
    pl.kernel,
    out_type=jax.ShapeDtypeStruct((NC * N_PAD, D), jnp.float32),
    mesh=_mesh,
    scratch_types=(
        pltpu.VMEM((CHUNK,), jnp.int32),
        pltpu.VMEM((CHUNK,), jnp.int32),
        pltpu.VMEM((CHUNK,), jnp.int32),
        pltpu.VMEM((CHUNK,), jnp.int32),
        pltpu.VMEM((CHUNK, D), jnp.float32),
        pltpu.VMEM((CHUNK, D), jnp.float32),
        pltpu.VMEM_SHARED((N_PAD, D), jnp.float32),
        pltpu.SemaphoreType.DMA,
        pltpu.SemaphoreType.DMA,
        pltpu.SemaphoreType.DMA,
        pltpu.SemaphoreType.DMA,
    ),
    compiler_params=_sc_params,
)
def _k4(msg_hbm, nidx_hbm, eidx_hbm, out_hbm,
        na, ea, nb, eb, rows_a, rows_b, out_acc, sia, sib, sga, sgb):
    cid = lax.axis_index("c")
    sid = lax.axis_index("s")

    _zero_2d(rows_a, CHUNK, D)
    for j in range(STAGE_STEPS):
        pltpu.sync_copy(
            rows_a, out_acc.at[pl.ds(sid * ROWS_PER_TILE + j * CHUNK, CHUNK)])
    plsc.subcore_barrier()

    base = cid * (INC_PAD // NC) + sid * PAIRS_HALF
    _gs_pipeline(msg_hbm, eidx_hbm, nidx_hbm, base, CHUNKS_HALF,
                 na, ea, nb, eb, rows_a, rows_b, sia, sib, sga, sgb, out_acc)

    plsc.subcore_barrier()

    for j in range(STAGE_STEPS):
        r = sid * ROWS_PER_TILE + j * CHUNK
        pltpu.sync_copy(out_acc.at[pl.ds(r, CHUNK)], rows_a)
        pltpu.sync_copy(rows_a, out_hbm.at[pl.ds(cid * N_PAD + r, CHUNK)])


def _k0_body(x_ref, s_ref, w_ref, b_ref, o_ref):
    xs = x_ref[...] * s_ref[...]
    o_ref[...] = jnp.dot(xs, w_ref[...],
                         preferred_element_type=jnp.float32) + b_ref[...]


def _k3_body(esum_ref, cnt_ref, den_ref, deg_ref, msg_ref, isd_ref):
    esum = esum_ref[0] + esum_ref[1]
    cnt_safe = jnp.maximum(cnt_ref[...], 1.0)
    h = esum / cnt_safe
    d_e = jnp.maximum(den_ref[...] / cnt_safe, 1.0)
    msg_ref[...] = h * lax.rsqrt(d_e)
    isd_ref[...] = lax.rsqrt(jnp.maximum(deg_ref[...], 1.0))


def _k5_body(p_ref, isd_ref, o_ref):
    o_ref[...] = (p_ref[0] + p_ref[1]) * isd_ref[...]


def kernel(node_features, incidence_nodes, incidence_edges,
           perturbation_mask, W, b):
    f32 = jnp.float32
    x_pad = jnp.pad(node_features, ((0, N_PAD - N), (0, 0)))
    scale = (~perturbation_mask).astype(f32)[:, None]
    scale_pad = jnp.pad(scale, ((0, N_PAD - N), (0, 0)))
    b2 = b.reshape(1, D)

    pad = INC_PAD - INC
    nidx = jnp.concatenate(
        [incidence_nodes.astype(jnp.int32), jnp.full((pad,), N, jnp.int32)])
    eidx = jnp.concatenate(
        [incidence_edges.astype(jnp.int32), jnp.full((pad,), E, jnp.int32)])

    xt_pad = pl.pallas_call(
        _k0_body,
        grid=(5,),
        in_specs=[
            pl.BlockSpec((2048, D), lambda i: (i, 0)),
            pl.BlockSpec((2048, 1), lambda i: (i, 0)),
            pl.BlockSpec((D, D), lambda i: (0, 0)),
            pl.BlockSpec((1, D), lambda i: (0, 0)),
        ],
        out_specs=pl.BlockSpec((2048, D), lambda i: (i, 0)),
        out_shape=jax.ShapeDtypeStruct((N_PAD, D), f32),
    )(x_pad, scale_pad, W, b2)

    esum_p, deg, cnt, den = _k1(xt_pad, nidx, eidx)
    esum_p = esum_p.reshape(NC, E_PAD, D)

    msg, isd = pl.pallas_call(
        _k3_body,
        grid=(1,),
        in_specs=[
            pl.BlockSpec((NC, E_PAD, D), lambda i: (0, 0, 0)),
            pl.BlockSpec((E_PAD, 1), lambda i: (0, 0)),
            pl.BlockSpec((E_PAD, 1), lambda i: (0, 0)),
            pl.BlockSpec((N_PAD, 1), lambda i: (0, 0)),
        ],
        out_specs=[
            pl.BlockSpec((E_PAD, D), lambda i: (0, 0)),
            pl.BlockSpec((N_PAD, 1), lambda i: (0, 0)),
        ],
        out_shape=[
            jax.ShapeDtypeStruct((E_PAD, D), f32),
            jax.ShapeDtypeStruct((N_PAD, 1), f32),
        ],
    )(esum_p, cnt[:, None], den[:, None], deg[:, None])

    out_p = _k4(msg, nidx, eidx).reshape(NC, N_PAD, D)

    out = pl.pallas_call(
        _k5_body,
        grid=(5,),
        in_specs=[
            pl.BlockSpec((NC, 2000, D), lambda i: (0, i, 0)),
            pl.BlockSpec((2000, 1), lambda i: (i, 0)),
        ],
        out_specs=pl.BlockSpec((2000, D), lambda i: (i, 0)),
        out_shape=jax.ShapeDtypeStruct((N, D), f32),
    )(out_p, isd)
    return out


# speedup vs baseline: 20.9429x; 3.3019x over previous
import functools

import jax
import jax.numpy as jnp
from jax import lax
from jax.experimental import pallas as pl
from jax.experimental.pallas import tpu as pltpu
from jax.experimental.pallas import tpu_sc as plsc

N = 10000
E = 2000
INC = 320000
D = 128

N_PAD = 10240
E_PAD = 2048
CHUNK = 128
NC, NS = 2, 16
NW = NC * NS
INC_PAD = 327680
PAIRS_ALL = INC_PAD // NS
CHUNKS_ALL = PAIRS_ALL // CHUNK
PAIRS_HALF = INC_PAD // NW
CHUNKS_HALF = PAIRS_HALF // CHUNK

_mesh = plsc.VectorSubcoreMesh(core_axis_name="c", subcore_axis_name="s")
_sc_params = pltpu.CompilerParams(needs_layout_passes=False)


def _zero_1d(ref, nwords):
    def body(i, _):
        ref[pl.ds(i * 16, 16)] = jnp.zeros((16,), jnp.float32)
        return 0
    lax.fori_loop(0, nwords // 16, body, 0)


def _zero_2d(ref, rows, cols):
    def body(i, _):
        r = i // (cols // 16)
        c = (i % (cols // 16)) * 16
        ref[r, pl.ds(c, 16)] = jnp.zeros((16,), jnp.float32)
        return 0
    lax.fori_loop(0, rows * (cols // 16), body, 0)


def _vadd_1d(dst, src, nwords):
    def body(i, _):
        s = pl.ds(i * 16, 16)
        dst[s] = dst[s] + src[s]
        return 0
    lax.fori_loop(0, nwords // 16, body, 0)


def _tree_reduce(sid, vec, tmp, shared, nwords):
    pltpu.sync_copy(vec, shared.at[sid])
    plsc.subcore_barrier()
    for r in (8, 4, 2, 1):
        @pl.when(sid < r)
        def _():
            pltpu.sync_copy(shared.at[sid + r], tmp)
            _vadd_1d(vec, tmp, nwords)
            pltpu.sync_copy(vec, shared.at[sid])
        plsc.subcore_barrier()


def _gs_pipeline(src_hbm, nidx_hbm, eidx_hbm, base, nchunks,
                 na, ea, nb, eb, rows_a, rows_b,
                 sia, sib, sga, sgb, acc):
    assert nchunks % 2 == 0

    def ld(n_bank, e_bank, sem, c):
        off = base + c * CHUNK
        pltpu.async_copy(nidx_hbm.at[pl.ds(off, CHUNK)], n_bank, sem)
        pltpu.async_copy(eidx_hbm.at[pl.ds(off, CHUNK)], e_bank, sem)

    def ld_wait(n_bank, e_bank, sem, c):
        off = base + c * CHUNK
        pltpu.make_async_copy(nidx_hbm.at[pl.ds(off, CHUNK)], n_bank, sem).wait()
        pltpu.make_async_copy(eidx_hbm.at[pl.ds(off, CHUNK)], e_bank, sem).wait()

    ld(na, ea, sia, 0)
    ld_wait(na, ea, sia, 0)
    pltpu.async_copy(src_hbm.at[na], rows_a, sga)
    ld(nb, eb, sib, 1)

    def body(i2, _):
        c0 = 2 * i2
        c1 = c0 + 1

        ld_wait(nb, eb, sib, c1)
        pltpu.async_copy(src_hbm.at[nb], rows_b, sgb)

        pltpu.make_async_copy(src_hbm.at[na], rows_a, sga).wait()
        pltpu.sync_copy(rows_a, acc.at[ea], add=True)

        @pl.when(c0 + 2 < nchunks)
        def _():
            ld(na, ea, sia, c0 + 2)
            ld_wait(na, ea, sia, c0 + 2)
            pltpu.async_copy(src_hbm.at[na], rows_a, sga)

        pltpu.make_async_copy(src_hbm.at[nb], rows_b, sgb).wait()
        pltpu.sync_copy(rows_b, acc.at[eb], add=True)

        @pl.when(c1 + 2 < nchunks)
        def _():
            ld(nb, eb, sib, c1 + 2)
        return 0
    lax.fori_loop(0, nchunks // 2, body, 0)


@functools.partial(
    pl.kernel,
    out_type=(
        jax.ShapeDtypeStruct((NC * E_PAD, D), jnp.float32),
        jax.ShapeDtypeStruct((N_PAD,), jnp.float32),
        jax.ShapeDtypeStruct((E_PAD,), jnp.float32),
        jax.ShapeDtypeStruct((E_PAD,), jnp.float32),
    ),
    mesh=_mesh,
    scratch_types=(
        pltpu.VMEM((PAIRS_ALL,), jnp.int32),
        pltpu.VMEM((PAIRS_ALL,), jnp.int32),
        pltpu.VMEM((CHUNK,), jnp.int32),
        pltpu.VMEM((CHUNK,), jnp.int32),
        pltpu.VMEM((CHUNK,), jnp.int32),
        pltpu.VMEM((CHUNK,), jnp.int32),
        pltpu.VMEM((CHUNK, D), jnp.float32),
        pltpu.VMEM((CHUNK, D), jnp.float32),
        pltpu.VMEM((N_PAD,), jnp.float32),
        pltpu.VMEM((N_PAD,), jnp.float32),
        pltpu.VMEM((E_PAD,), jnp.float32),
        pltpu.VMEM((E_PAD,), jnp.float32),
        pltpu.VMEM_SHARED((NS, N_PAD), jnp.float32),
        pltpu.VMEM_SHARED((NS, E_PAD), jnp.float32),
        pltpu.VMEM_SHARED((E_PAD, D), jnp.float32),
        pltpu.SemaphoreType.DMA,
        pltpu.SemaphoreType.DMA,
        pltpu.SemaphoreType.DMA,
        pltpu.SemaphoreType.DMA,
    ),
    compiler_params=_sc_params,
)
def _k1(xt_hbm, nidx_hbm, eidx_hbm,
        esum_out, deg_out, cnt_out, den_out,
        nidx_all, eidx_all, na, ea, nb, eb, rows_a, rows_b,
        deg_v, deg_tmp, ev_v, ev_tmp,
        sh_deg, sh_edge, e_acc, sia, sib, sga, sgb):
    cid = lax.axis_index("c")
    sid = lax.axis_index("s")
    ones = jnp.ones((16,), jnp.float32)

    pltpu.sync_copy(nidx_hbm.at[pl.ds(sid * PAIRS_ALL, PAIRS_ALL)], nidx_all)
    pltpu.sync_copy(eidx_hbm.at[pl.ds(sid * PAIRS_ALL, PAIRS_ALL)], eidx_all)

    _zero_1d(deg_v, N_PAD)
    _zero_1d(ev_v, E_PAD)
    _zero_2d(rows_a, CHUNK, D)
    pltpu.sync_copy(rows_a, e_acc.at[pl.ds(sid * CHUNK, CHUNK)])

    def vec_a(k, _):
        nv = nidx_all[pl.ds(k * 16, 16)]
        evi = eidx_all[pl.ds(k * 16, 16)]
        plsc.addupdate_scatter(deg_v, [nv], ones)
        plsc.addupdate_scatter(ev_v, [evi], ones)
        return 0
    lax.fori_loop(0, PAIRS_ALL // 16, vec_a, 0)

    _tree_reduce(sid, deg_v, deg_tmp, sh_deg, N_PAD)
    _tree_reduce(sid, ev_v, ev_tmp, sh_edge, E_PAD)

    @pl.when(jnp.logical_and(cid == 0, sid == 0))
    def _():
        pltpu.sync_copy(deg_v, deg_out)
        pltpu.sync_copy(ev_v, cnt_out)

    plsc.subcore_barrier()
    pltpu.sync_copy(sh_deg.at[0], deg_v)

    _zero_1d(ev_v, E_PAD)

    def vec_b(k, _):
        nv = nidx_all[pl.ds(k * 16, 16)]
        evi = eidx_all[pl.ds(k * 16, 16)]
        dv = plsc.load_gather(deg_v, [nv])
        plsc.addupdate_scatter(ev_v, [evi], dv)
        return 0
    lax.fori_loop(0, PAIRS_ALL // 16, vec_b, 0)

    _tree_reduce(sid, ev_v, ev_tmp, sh_edge, E_PAD)

    @pl.when(jnp.logical_and(cid == 0, sid == 0))
    def _():
        pltpu.sync_copy(ev_v, den_out)

    base = cid * (INC_PAD // NC) + sid * PAIRS_HALF
    _gs_pipeline(xt_hbm, nidx_hbm, eidx_hbm, base, CHUNKS_HALF,
                 na, ea, nb, eb, rows_a, rows_b, sia, sib, sga, sgb, e_acc)

    plsc.subcore_barrier()

    pltpu.sync_copy(e_acc.at[pl.ds(sid * CHUNK, CHUNK)], rows_a)
    pltpu.sync_copy(rows_a, esum_out.at[pl.ds(cid * E_PAD + sid * CHUNK, CHUNK)])


ROWS_PER_TILE = N_PAD // NS
STAGE_STEPS = ROWS_PER_TILE // CHUNK


@functools.partial(
    pl.kernel,
    out_type=jax.ShapeDtypeStruct((NC * N_PAD, D), jnp.float32),
    mesh=_mesh,
    scratch_types=(
        pltpu.VMEM((CHUNK,), jnp.int32),
        pltpu.VMEM((CHUNK,), jnp.int32),
        pltpu.VMEM((CHUNK,), jnp.int32),
        pltpu.VMEM((CHUNK,), jnp.int32),
        pltpu.VMEM((CHUNK, D), jnp.float32),
        pltpu.VMEM((CHUNK, D), jnp.float32),
        pltpu.VMEM_SHARED((N_PAD, D), jnp.float32),
        pltpu.SemaphoreType.DMA,
        pltpu.SemaphoreType.DMA,
        pltpu.SemaphoreType.DMA,
        pltpu.SemaphoreType.DMA,
    ),
    compiler_params=_sc_params,
)
def _k4(msg_hbm, nidx_hbm, eidx_hbm, out_hbm,
        na, ea, nb, eb, rows_a, rows_b, out_acc, sia, sib, sga, sgb):
    cid = lax.axis_index("c")
    sid = lax.axis_index("s")

    _zero_2d(rows_a, CHUNK, D)
    for j in range(STAGE_STEPS):
        pltpu.sync_copy(
            rows_a, out_acc.at[pl.ds(sid * ROWS_PER_TILE + j * CHUNK, CHUNK)])
    plsc.subcore_barrier()

    base = cid * (INC_PAD // NC) + sid * PAIRS_HALF
    _gs_pipeline(msg_hbm, eidx_hbm, nidx_hbm, base, CHUNKS_HALF,
                 na, ea, nb, eb, rows_a, rows_b, sia, sib, sga, sgb, out_acc)

    plsc.subcore_barrier()

    for j in range(STAGE_STEPS):
        r = sid * ROWS_PER_TILE + j * CHUNK
        pltpu.sync_copy(out_acc.at[pl.ds(r, CHUNK)], rows_a)
        pltpu.sync_copy(rows_a, out_hbm.at[pl.ds(cid * N_PAD + r, CHUNK)])


def _k0_body(x_ref, s_ref, w_ref, b_ref, o_ref):
    xs = x_ref[...] * s_ref[...]
    o_ref[...] = jnp.dot(xs, w_ref[...],
                         preferred_element_type=jnp.float32) + b_ref[...]


def _k3_body(esum_ref, cnt_ref, den_ref, msg_ref):
    esum = esum_ref[0] + esum_ref[1]
    cnt_safe = jnp.maximum(cnt_ref[...], 1.0)
    h = esum / cnt_safe
    d_e = jnp.maximum(den_ref[...] / cnt_safe, 1.0)
    msg_ref[...] = h * lax.rsqrt(d_e)


def _k5_body(p_ref, deg_ref, o_ref):
    isd = lax.rsqrt(jnp.maximum(deg_ref[...], 1.0))
    o_ref[...] = (p_ref[0] + p_ref[1]) * isd


def kernel(node_features, incidence_nodes, incidence_edges,
           perturbation_mask, W, b):
    f32 = jnp.float32
    x_pad = jnp.pad(node_features, ((0, N_PAD - N), (0, 0)))
    scale = (~perturbation_mask).astype(f32)[:, None]
    scale_pad = jnp.pad(scale, ((0, N_PAD - N), (0, 0)))
    b2 = b.reshape(1, D)

    pad = INC_PAD - INC
    r = jnp.arange(pad, dtype=jnp.int32)
    nidx = jnp.concatenate(
        [incidence_nodes.astype(jnp.int32), N + r % (N_PAD - N)])
    eidx = jnp.concatenate(
        [incidence_edges.astype(jnp.int32), E + r % (E_PAD - E)])

    xt_pad = pl.pallas_call(
        _k0_body,
        grid=(5,),
        in_specs=[
            pl.BlockSpec((2048, D), lambda i: (i, 0)),
            pl.BlockSpec((2048, 1), lambda i: (i, 0)),
            pl.BlockSpec((D, D), lambda i: (0, 0)),
            pl.BlockSpec((1, D), lambda i: (0, 0)),
        ],
        out_specs=pl.BlockSpec((2048, D), lambda i: (i, 0)),
        out_shape=jax.ShapeDtypeStruct((N_PAD, D), f32),
    )(x_pad, scale_pad, W, b2)

    esum_p, deg, cnt, den = _k1(xt_pad, nidx, eidx)

    msg = pl.pallas_call(
        _k3_body,
        grid=(1,),
        in_specs=[
            pl.BlockSpec((NC, E_PAD, D), lambda i: (0, 0, 0)),
            pl.BlockSpec((E_PAD, 1), lambda i: (0, 0)),
            pl.BlockSpec((E_PAD, 1), lambda i: (0, 0)),
        ],
        out_specs=pl.BlockSpec((E_PAD, D), lambda i: (0, 0)),
        out_shape=jax.ShapeDtypeStruct((E_PAD, D), f32),
    )(esum_p.reshape(NC, E_PAD, D), cnt[:, None], den[:, None])

    out_p = _k4(msg, nidx, eidx).reshape(NC, N_PAD, D)

    out = pl.pallas_call(
        _k5_body,
        grid=(5,),
        in_specs=[
            pl.BlockSpec((NC, 2000, D), lambda i: (0, i, 0)),
            pl.BlockSpec((2000, 1), lambda i: (i, 0)),
        ],
        out_specs=pl.BlockSpec((2000, D), lambda i: (i, 0)),
        out_shape=jax.ShapeDtypeStruct((N, D), f32),
    )(out_p, deg[:, None])
    return out


# speedup vs baseline: 21.3508x; 1.0195x over previous
import functools

import jax
import jax.numpy as jnp
from jax import lax
from jax.experimental import pallas as pl
from jax.experimental.pallas import tpu as pltpu
from jax.experimental.pallas import tpu_sc as plsc

N = 10000
E = 2000
INC = 320000
D = 128

N_PAD = 10240
E_PAD = 2048
CHUNK = 128
NC, NS = 2, 16
NW = NC * NS
INC_PAD = 327680
PAIRS_ALL = INC_PAD // NS
CHUNKS_ALL = PAIRS_ALL // CHUNK
PAIRS_HALF = INC_PAD // NW
CHUNKS_HALF = PAIRS_HALF // CHUNK

_mesh = plsc.VectorSubcoreMesh(core_axis_name="c", subcore_axis_name="s")
_sc_params = pltpu.CompilerParams(needs_layout_passes=False)


def _zero_1d(ref, nwords):
    def body(i, _):
        ref[pl.ds(i * 16, 16)] = jnp.zeros((16,), jnp.float32)
        return 0
    lax.fori_loop(0, nwords // 16, body, 0)


def _zero_2d(ref, rows, cols):
    def body(i, _):
        r = i // (cols // 16)
        c = (i % (cols // 16)) * 16
        ref[r, pl.ds(c, 16)] = jnp.zeros((16,), jnp.float32)
        return 0
    lax.fori_loop(0, rows * (cols // 16), body, 0)


def _vadd_1d(dst, src, nwords):
    def body(i, _):
        s = pl.ds(i * 16, 16)
        dst[s] = dst[s] + src[s]
        return 0
    lax.fori_loop(0, nwords // 16, body, 0)


def _tree_reduce(sid, vec, tmp, shared, nwords):
    pltpu.sync_copy(vec, shared.at[sid])
    plsc.subcore_barrier()
    for r in (8, 4, 2, 1):
        @pl.when(sid < r)
        def _():
            pltpu.sync_copy(shared.at[sid + r], tmp)
            _vadd_1d(vec, tmp, nwords)
            pltpu.sync_copy(vec, shared.at[sid])
        plsc.subcore_barrier()


def _gs_pipeline(src_hbm, nidx_hbm, eidx_hbm, base, nchunks,
                 na, ea, nb, eb, rows_a, rows_b,
                 sia, sib, sga, sgb, acc, filler=None):
    assert nchunks % 2 == 0

    def ld(n_bank, e_bank, sem, c):
        off = base + c * CHUNK
        pltpu.async_copy(nidx_hbm.at[pl.ds(off, CHUNK)], n_bank, sem)
        pltpu.async_copy(eidx_hbm.at[pl.ds(off, CHUNK)], e_bank, sem)

    def ld_wait(n_bank, e_bank, sem, c):
        off = base + c * CHUNK
        pltpu.make_async_copy(nidx_hbm.at[pl.ds(off, CHUNK)], n_bank, sem).wait()
        pltpu.make_async_copy(eidx_hbm.at[pl.ds(off, CHUNK)], e_bank, sem).wait()

    ld(na, ea, sia, 0)
    ld_wait(na, ea, sia, 0)
    pltpu.async_copy(src_hbm.at[na], rows_a, sga)
    ld(nb, eb, sib, 1)

    def body(i2, _):
        c0 = 2 * i2
        c1 = c0 + 1

        ld_wait(nb, eb, sib, c1)
        pltpu.async_copy(src_hbm.at[nb], rows_b, sgb)

        if filler is not None:
            filler(i2)

        pltpu.make_async_copy(src_hbm.at[na], rows_a, sga).wait()
        pltpu.sync_copy(rows_a, acc.at[ea], add=True)

        @pl.when(c0 + 2 < nchunks)
        def _():
            ld(na, ea, sia, c0 + 2)
            ld_wait(na, ea, sia, c0 + 2)
            pltpu.async_copy(src_hbm.at[na], rows_a, sga)

        pltpu.make_async_copy(src_hbm.at[nb], rows_b, sgb).wait()
        pltpu.sync_copy(rows_b, acc.at[eb], add=True)

        @pl.when(c1 + 2 < nchunks)
        def _():
            ld(nb, eb, sib, c1 + 2)
        return 0
    lax.fori_loop(0, nchunks // 2, body, 0)


@functools.partial(
    pl.kernel,
    out_type=(
        jax.ShapeDtypeStruct((NC * E_PAD, D), jnp.float32),
        jax.ShapeDtypeStruct((N_PAD,), jnp.float32),
        jax.ShapeDtypeStruct((E_PAD,), jnp.float32),
        jax.ShapeDtypeStruct((E_PAD,), jnp.float32),
    ),
    mesh=_mesh,
    scratch_types=(
        pltpu.VMEM((PAIRS_ALL,), jnp.int32),
        pltpu.VMEM((PAIRS_ALL,), jnp.int32),
        pltpu.VMEM((CHUNK,), jnp.int32),
        pltpu.VMEM((CHUNK,), jnp.int32),
        pltpu.VMEM((CHUNK,), jnp.int32),
        pltpu.VMEM((CHUNK,), jnp.int32),
        pltpu.VMEM((CHUNK, D), jnp.float32),
        pltpu.VMEM((CHUNK, D), jnp.float32),
        pltpu.VMEM((N_PAD,), jnp.float32),
        pltpu.VMEM((N_PAD,), jnp.float32),
        pltpu.VMEM((E_PAD,), jnp.float32),
        pltpu.VMEM((E_PAD,), jnp.float32),
        pltpu.VMEM_SHARED((NS, N_PAD), jnp.float32),
        pltpu.VMEM_SHARED((NS, E_PAD), jnp.float32),
        pltpu.VMEM_SHARED((E_PAD, D), jnp.float32),
        pltpu.SemaphoreType.DMA,
        pltpu.SemaphoreType.DMA,
        pltpu.SemaphoreType.DMA,
        pltpu.SemaphoreType.DMA,
    ),
    compiler_params=_sc_params,
)
def _k1(xt_hbm, nidx_hbm, eidx_hbm,
        esum_out, deg_out, cnt_out, den_out,
        nidx_all, eidx_all, na, ea, nb, eb, rows_a, rows_b,
        deg_v, deg_tmp, ev_v, ev_tmp,
        sh_deg, sh_edge, e_acc, sia, sib, sga, sgb):
    cid = lax.axis_index("c")
    sid = lax.axis_index("s")
    ones = jnp.ones((16,), jnp.float32)

    pltpu.sync_copy(nidx_hbm.at[pl.ds(sid * PAIRS_ALL, PAIRS_ALL)], nidx_all)
    pltpu.sync_copy(eidx_hbm.at[pl.ds(sid * PAIRS_ALL, PAIRS_ALL)], eidx_all)

    _zero_1d(deg_v, N_PAD)
    _zero_1d(ev_v, E_PAD)
    _zero_2d(rows_a, CHUNK, D)
    pltpu.sync_copy(rows_a, e_acc.at[pl.ds(sid * CHUNK, CHUNK)])
    plsc.subcore_barrier()

    def vec_a(k):
        nv = nidx_all[pl.ds(k * 16, 16)]
        evi = eidx_all[pl.ds(k * 16, 16)]
        plsc.addupdate_scatter(deg_v, [nv], ones)
        plsc.addupdate_scatter(ev_v, [evi], ones)

    A_PER_BODY = (PAIRS_ALL // 16) // (CHUNKS_HALF // 2)

    def filler(i2):
        for t in range(A_PER_BODY):
            vec_a(i2 * A_PER_BODY + t)

    base = cid * (INC_PAD // NC) + sid * PAIRS_HALF
    _gs_pipeline(xt_hbm, nidx_hbm, eidx_hbm, base, CHUNKS_HALF,
                 na, ea, nb, eb, rows_a, rows_b, sia, sib, sga, sgb, e_acc,
                 filler=filler)

    _tree_reduce(sid, deg_v, deg_tmp, sh_deg, N_PAD)
    _tree_reduce(sid, ev_v, ev_tmp, sh_edge, E_PAD)

    @pl.when(jnp.logical_and(cid == 0, sid == 0))
    def _():
        pltpu.sync_copy(deg_v, deg_out)
        pltpu.sync_copy(ev_v, cnt_out)

    plsc.subcore_barrier()
    pltpu.sync_copy(sh_deg.at[0], deg_v)

    _zero_1d(ev_v, E_PAD)

    def vec_b(k, _):
        nv = nidx_all[pl.ds(k * 16, 16)]
        evi = eidx_all[pl.ds(k * 16, 16)]
        dv = plsc.load_gather(deg_v, [nv])
        plsc.addupdate_scatter(ev_v, [evi], dv)
        return 0
    lax.fori_loop(0, PAIRS_ALL // 16, vec_b, 0)

    _tree_reduce(sid, ev_v, ev_tmp, sh_edge, E_PAD)

    @pl.when(jnp.logical_and(cid == 0, sid == 0))
    def _():
        pltpu.sync_copy(ev_v, den_out)

    plsc.subcore_barrier()

    pltpu.sync_copy(e_acc.at[pl.ds(sid * CHUNK, CHUNK)], rows_a)
    pltpu.sync_copy(rows_a, esum_out.at[pl.ds(cid * E_PAD + sid * CHUNK, CHUNK)])


ROWS_PER_TILE = N_PAD // NS
STAGE_STEPS = ROWS_PER_TILE // CHUNK


@functools.partial(
    pl.kernel,
    out_type=jax.ShapeDtypeStruct((NC * N_PAD, D), jnp.float32),
    mesh=_mesh,
    scratch_types=(
        pltpu.VMEM((CHUNK,), jnp.int32),
        pltpu.VMEM((CHUNK,), jnp.int32),
        pltpu.VMEM((CHUNK,), jnp.int32),
        pltpu.VMEM((CHUNK,), jnp.int32),
        pltpu.VMEM((CHUNK, D), jnp.float32),
        pltpu.VMEM((CHUNK, D), jnp.float32),
        pltpu.VMEM_SHARED((N_PAD, D), jnp.float32),
        pltpu.SemaphoreType.DMA,
        pltpu.SemaphoreType.DMA,
        pltpu.SemaphoreType.DMA,
        pltpu.SemaphoreType.DMA,
    ),
    compiler_params=_sc_params,
)
def _k4(msg_hbm, nidx_hbm, eidx_hbm, out_hbm,
        na, ea, nb, eb, rows_a, rows_b, out_acc, sia, sib, sga, sgb):
    cid = lax.axis_index("c")
    sid = lax.axis_index("s")

    _zero_2d(rows_a, CHUNK, D)
    for j in range(STAGE_STEPS):
        pltpu.sync_copy(
            rows_a, out_acc.at[pl.ds(sid * ROWS_PER_TILE + j * CHUNK, CHUNK)])
    plsc.subcore_barrier()

    base = cid * (INC_PAD // NC) + sid * PAIRS_HALF
    _gs_pipeline(msg_hbm, eidx_hbm, nidx_hbm, base, CHUNKS_HALF,
                 na, ea, nb, eb, rows_a, rows_b, sia, sib, sga, sgb, out_acc)

    plsc.subcore_barrier()

    for j in range(STAGE_STEPS):
        r = sid * ROWS_PER_TILE + j * CHUNK
        pltpu.sync_copy(out_acc.at[pl.ds(r, CHUNK)], rows_a)
        pltpu.sync_copy(rows_a, out_hbm.at[pl.ds(cid * N_PAD + r, CHUNK)])


def _k0_body(x_ref, s_ref, w_ref, b_ref, o_ref):
    xs = x_ref[...] * s_ref[...]
    o_ref[...] = jnp.dot(xs, w_ref[...],
                         preferred_element_type=jnp.float32) + b_ref[...]


def _k3_body(esum_ref, cnt_ref, den_ref, msg_ref):
    esum = esum_ref[0] + esum_ref[1]
    cnt_safe = jnp.maximum(cnt_ref[...], 1.0)
    h = esum / cnt_safe
    d_e = jnp.maximum(den_ref[...] / cnt_safe, 1.0)
    msg_ref[...] = h * lax.rsqrt(d_e)


def _k5_body(p_ref, deg_ref, o_ref):
    isd = lax.rsqrt(jnp.maximum(deg_ref[...], 1.0))
    o_ref[...] = (p_ref[0] + p_ref[1]) * isd


def kernel(node_features, incidence_nodes, incidence_edges,
           perturbation_mask, W, b):
    f32 = jnp.float32
    x_pad = jnp.pad(node_features, ((0, N_PAD - N), (0, 0)))
    scale = (~perturbation_mask).astype(f32)[:, None]
    scale_pad = jnp.pad(scale, ((0, N_PAD - N), (0, 0)))
    b2 = b.reshape(1, D)

    pad = INC_PAD - INC
    r = jnp.arange(pad, dtype=jnp.int32)
    nidx = jnp.concatenate(
        [incidence_nodes.astype(jnp.int32), N + r % (N_PAD - N)])
    eidx = jnp.concatenate(
        [incidence_edges.astype(jnp.int32), E + r % (E_PAD - E)])

    xt_pad = pl.pallas_call(
        _k0_body,
        grid=(5,),
        in_specs=[
            pl.BlockSpec((2048, D), lambda i: (i, 0)),
            pl.BlockSpec((2048, 1), lambda i: (i, 0)),
            pl.BlockSpec((D, D), lambda i: (0, 0)),
            pl.BlockSpec((1, D), lambda i: (0, 0)),
        ],
        out_specs=pl.BlockSpec((2048, D), lambda i: (i, 0)),
        out_shape=jax.ShapeDtypeStruct((N_PAD, D), f32),
    )(x_pad, scale_pad, W, b2)

    esum_p, deg, cnt, den = _k1(xt_pad, nidx, eidx)

    msg = pl.pallas_call(
        _k3_body,
        grid=(1,),
        in_specs=[
            pl.BlockSpec((NC, E_PAD, D), lambda i: (0, 0, 0)),
            pl.BlockSpec((E_PAD, 1), lambda i: (0, 0)),
            pl.BlockSpec((E_PAD, 1), lambda i: (0, 0)),
        ],
        out_specs=pl.BlockSpec((E_PAD, D), lambda i: (0, 0)),
        out_shape=jax.ShapeDtypeStruct((E_PAD, D), f32),
    )(esum_p.reshape(NC, E_PAD, D), cnt[:, None], den[:, None])

    out_p = _k4(msg, nidx, eidx).reshape(NC, N_PAD, D)

    out = pl.pallas_call(
        _k5_body,
        grid=(5,),
        in_specs=[
            pl.BlockSpec((NC, 2000, D), lambda i: (0, i, 0)),
            pl.BlockSpec((2000, 1), lambda i: (i, 0)),
        ],
        out_specs=pl.BlockSpec((2000, D), lambda i: (i, 0)),
        out_shape=jax.ShapeDtypeStruct((N, D), f32),
    )(out_p, deg[:, None])
    return out
